# Initial kernel scaffold; baseline (speedup 1.0000x reference)
import functools

import jax
import jax.numpy as jnp
from jax import lax
from jax.experimental import pallas as pl
from jax.experimental.pallas import tpu as pltpu
from jax.experimental.pallas import tpu_sc as plsc

N = 10000
D = 128
EDGE_DIM = 16
E = 320000
N_LAYERS = 3

NW = 32
CHUNK = 128
NCHUNK = 80
EPW = NCHUNK * CHUNK
E_PAD = NW * EPW
PAD = E_PAD - E
N_PAD = 10240

NB = 400
EB = 512

_mesh = plsc.VectorSubcoreMesh(core_axis_name="c", subcore_axis_name="s")


@functools.partial(
    pl.kernel,
    mesh=_mesh,
    out_type=jax.ShapeDtypeStruct((NW, NCHUNK, CHUNK, D), jnp.float32),
    scratch_types=[
        pltpu.VMEM((NCHUNK, CHUNK), jnp.int32),
        pltpu.VMEM((CHUNK, D), jnp.float32),
        pltpu.VMEM((CHUNK, D), jnp.float32),
        pltpu.SemaphoreType.DMA,
        pltpu.SemaphoreType.DMA,
    ],
)
def _sc_gather(table_hbm, idx_hbm, out_hbm, idx_v, buf0, buf1, sem0, sem1):
    wid = lax.axis_index("s") * 2 + lax.axis_index("c")
    pltpu.sync_copy(idx_hbm.at[wid], idx_v)

    def body(i, carry):
        j0 = 2 * i
        j1 = j0 + 1
        c0 = pltpu.async_copy(table_hbm.at[idx_v.at[j0]], buf0, sem0)
        c1 = pltpu.async_copy(table_hbm.at[idx_v.at[j1]], buf1, sem1)
        c0.wait()
        pltpu.sync_copy(buf0, out_hbm.at[wid, j0])
        c1.wait()
        pltpu.sync_copy(buf1, out_hbm.at[wid, j1])
        return carry

    lax.fori_loop(0, NCHUNK // 2, body, 0)


@functools.partial(
    pl.kernel,
    mesh=_mesh,
    out_type=jax.ShapeDtypeStruct((2, N, D), jnp.float32),
    scratch_types=[
        pltpu.VMEM((NCHUNK, CHUNK), jnp.int32),
        pltpu.VMEM((CHUNK, D), jnp.float32),
        pltpu.VMEM((CHUNK, D), jnp.float32),
        pltpu.VMEM_SHARED((N_PAD, D), jnp.float32),
        pltpu.SemaphoreType.DMA,
        pltpu.SemaphoreType.DMA,
    ],
)
def _sc_scatter(m_hbm, idx_hbm, z_hbm, out_hbm,
                idx_v, buf0, buf1, acc, sem0, sem1):
    c = lax.axis_index("c")
    s = lax.axis_index("s")
    wid = s * 2 + c
    rows_per_sub = N_PAD // 16

    pltpu.sync_copy(z_hbm, buf0)

    def zbody(t, carry):
        pltpu.sync_copy(buf0, acc.at[pl.ds(s * rows_per_sub + t * CHUNK, CHUNK)])
        return carry

    lax.fori_loop(0, rows_per_sub // CHUNK, zbody, 0)
    pltpu.sync_copy(idx_hbm.at[wid], idx_v)
    plsc.subcore_barrier()

    def body(i, carry):
        j0 = 2 * i
        j1 = j0 + 1
        c0 = pltpu.async_copy(m_hbm.at[wid, j0], buf0, sem0)
        c1 = pltpu.async_copy(m_hbm.at[wid, j1], buf1, sem1)
        c0.wait()
        pltpu.sync_copy(buf0, acc.at[idx_v.at[j0]], add=True)
        c1.wait()
        pltpu.sync_copy(buf1, acc.at[idx_v.at[j1]], add=True)
        return carry

    lax.fori_loop(0, NCHUNK // 2, body, 0)
    plsc.subcore_barrier()

    out_rows = N // 16
    pltpu.sync_copy(acc.at[pl.ds(s * out_rows, out_rows)],
                    out_hbm.at[c, pl.ds(s * out_rows, out_rows)])


def _full(shape):
    return pl.BlockSpec(shape, lambda i: (0,) * len(shape))


def _node_tc(x, mp, ln_g, ln_b, w1h_msg):
    def body(x_ref, w1, b1, w2, b2, w3, b3, g, b, wm, h_ref, p_ref):
        xv = jnp.nan_to_num(x_ref[...], nan=0.0, posinf=0.0, neginf=0.0)
        h = jnp.maximum(xv @ w1[...] + b1[...], 0.0)
        h = jnp.maximum(h @ w2[...] + b2[...], 0.0)
        h = h @ w3[...] + b3[...]
        mu = jnp.mean(h, axis=-1, keepdims=True)
        var = jnp.mean((h - mu) ** 2, axis=-1, keepdims=True)
        hn = (h - mu) * lax.rsqrt(var + 1e-5) * g[...] + b[...]
        h_ref[...] = hn
        p_ref[...] = hn @ wm[...]

    return pl.pallas_call(
        body,
        grid=(N // NB,),
        in_specs=[
            pl.BlockSpec((NB, D), lambda i: (i, 0)),
            _full((D, D)), _full((1, D)), _full((D, D)), _full((1, D)),
            _full((D, D)), _full((1, D)), _full((1, D)), _full((1, D)),
            _full((D, D)),
        ],
        out_specs=[
            pl.BlockSpec((NB, D), lambda i: (i, 0)),
            pl.BlockSpec((NB, D), lambda i: (i, 0)),
        ],
        out_shape=[
            jax.ShapeDtypeStruct((N, D), jnp.float32),
            jax.ShapeDtypeStruct((N, D), jnp.float32),
        ],
    )(x, mp['W1'], mp['b1'].reshape(1, D), mp['W2'], mp['b2'].reshape(1, D),
      mp['W3'], mp['b3'].reshape(1, D), ln_g.reshape(1, D), ln_b.reshape(1, D),
      w1h_msg)


def _ea_tc(ea_pad, w1e, b1):
    def body(ea_ref, w_ref, b_ref, out_ref):
        ea = jnp.nan_to_num(ea_ref[...], nan=0.0, posinf=0.0, neginf=0.0)
        out_ref[...] = ea @ w_ref[...] + b_ref[...]

    return pl.pallas_call(
        body,
        grid=(E_PAD // EB,),
        in_specs=[
            pl.BlockSpec((EB, EDGE_DIM), lambda i: (i, 0)),
            _full((EDGE_DIM, D)), _full((1, D)),
        ],
        out_specs=pl.BlockSpec((EB, D), lambda i: (i, 0)),
        out_shape=jax.ShapeDtypeStruct((E_PAD, D), jnp.float32),
    )(ea_pad, w1e, b1.reshape(1, D))


def _msg_tc(g_arr, ea_arr, w2, b2, w3, b3):
    def body(g_ref, ea_ref, w2r, b2r, w3r, b3r, m_ref):
        h = jnp.maximum(g_ref[...] + ea_ref[...], 0.0)
        h = jnp.maximum(h @ w2r[...] + b2r[...], 0.0)
        m_ref[...] = h @ w3r[...] + b3r[...]

    return pl.pallas_call(
        body,
        grid=(E_PAD // EB,),
        in_specs=[
            pl.BlockSpec((EB, D), lambda i: (i, 0)),
            pl.BlockSpec((EB, D), lambda i: (i, 0)),
            _full((D, D)), _full((1, D)), _full((D, D)), _full((1, D)),
        ],
        out_specs=pl.BlockSpec((EB, D), lambda i: (i, 0)),
        out_shape=jax.ShapeDtypeStruct((E_PAD, D), jnp.float32),
    )(g_arr, ea_arr, w2, b2.reshape(1, D), w3, b3.reshape(1, D))


def _update_tc(h, p0, p1, up, ln_g, ln_b, w1h_msg, compute_mean):
    nb = N // NB
    w1 = up['W1']

    def body(h_ref, p0_ref, p1_ref, w1h, w1a, b1, w2, b2, w3, b3, g, b, wm,
             *outs):
        agg = p0_ref[...] + p1_ref[...]
        hv = h_ref[...]
        u = jnp.maximum(hv @ w1h[...] + agg @ w1a[...] + b1[...], 0.0)
        u = jnp.maximum(u @ w2[...] + b2[...], 0.0)
        u = u @ w3[...] + b3[...]
        hh = hv + u
        mu = jnp.mean(hh, axis=-1, keepdims=True)
        var = jnp.mean((hh - mu) ** 2, axis=-1, keepdims=True)
        hn = (hh - mu) * lax.rsqrt(var + 1e-5) * g[...] + b[...]
        outs[0][...] = hn
        outs[1][...] = hn @ wm[...]
        if compute_mean:
            i = pl.program_id(0)
            gacc = outs[2]

            @pl.when(i == 0)
            def _():
                gacc[...] = jnp.zeros_like(gacc)

            gacc[...] += jnp.sum(hn, axis=0, keepdims=True)

            @pl.when(i == nb - 1)
            def _():
                gacc[...] = gacc[...] * (1.0 / N)

    out_specs = [
        pl.BlockSpec((NB, D), lambda i: (i, 0)),
        pl.BlockSpec((NB, D), lambda i: (i, 0)),
    ]
    out_shape = [
        jax.ShapeDtypeStruct((N, D), jnp.float32),
        jax.ShapeDtypeStruct((N, D), jnp.float32),
    ]
    if compute_mean:
        out_specs.append(_full((1, D)))
        out_shape.append(jax.ShapeDtypeStruct((1, D), jnp.float32))

    return pl.pallas_call(
        body,
        grid=(nb,),
        in_specs=[
            pl.BlockSpec((NB, D), lambda i: (i, 0)),
            pl.BlockSpec((NB, D), lambda i: (i, 0)),
            pl.BlockSpec((NB, D), lambda i: (i, 0)),
            _full((D, D)), _full((D, D)), _full((1, D)), _full((D, D)),
            _full((1, D)), _full((D, D)), _full((1, D)), _full((1, D)),
            _full((1, D)), _full((D, D)),
        ],
        out_specs=out_specs,
        out_shape=out_shape,
    )(h, p0, p1, w1[:D], w1[D:], up['b1'].reshape(1, D), up['W2'],
      up['b2'].reshape(1, D), up['W3'], up['b3'].reshape(1, D),
      ln_g.reshape(1, D), ln_b.reshape(1, D), w1h_msg)


def kernel(node_x, edge_index, edge_attr, params):
    node_x = node_x.astype(jnp.float32)
    edge_attr = edge_attr.astype(jnp.float32)
    src = edge_index[0].astype(jnp.int32)
    dst = edge_index[1].astype(jnp.int32)

    src_r = jnp.concatenate([src, jnp.zeros((PAD,), jnp.int32)]
                            ).reshape(NW, NCHUNK, CHUNK)
    dst_r = jnp.concatenate([dst, jnp.full((PAD,), N, jnp.int32)]
                            ).reshape(NW, NCHUNK, CHUNK)
    ea_pad = jnp.concatenate(
        [edge_attr, jnp.zeros((PAD, EDGE_DIM), jnp.float32)])

    mp = params['msg_mlp']
    w1h_msg = mp['W1'][:D]
    ln_g, ln_b = params['ln_g'], params['ln_b']

    EA = _ea_tc(ea_pad, mp['W1'][D:], mp['b1'])
    H, P = _node_tc(node_x, params['node_mlp'], ln_g, ln_b, w1h_msg)
    zeros_blk = jnp.zeros((CHUNK, D), jnp.float32)

    gsum = None
    for layer in range(N_LAYERS):
        G = _sc_gather(P, src_r).reshape(E_PAD, D)
        M = _msg_tc(G, EA, mp['W2'], mp['b2'], mp['W3'], mp['b3'])
        part = _sc_scatter(M.reshape(NW, NCHUNK, CHUNK, D), dst_r, zeros_blk)
        last = layer == N_LAYERS - 1
        if last:
            H, P, gsum = _update_tc(H, part[0], part[1], params['up_mlp'],
                                    ln_g, ln_b, w1h_msg, True)
        else:
            H, P = _update_tc(H, part[0], part[1], params['up_mlp'],
                              ln_g, ln_b, w1h_msg, False)

    return (H, gsum.reshape(D))


# speedup vs baseline: 1.2647x; 1.2647x over previous
import functools

import jax
import jax.numpy as jnp
from jax import lax
from jax.experimental import pallas as pl
from jax.experimental.pallas import tpu as pltpu
from jax.experimental.pallas import tpu_sc as plsc

N = 10000
D = 128
EDGE_DIM = 16
E = 320000
N_LAYERS = 3

NW = 32
CHUNK = 128
NCHUNK = 80
EPW = NCHUNK * CHUNK
E_PAD = NW * EPW
PAD = E_PAD - E
N_PAD = 10240

NB = 400
EB = 512


@functools.cache
def _sc_gather_kernel():
    mesh = plsc.VectorSubcoreMesh(core_axis_name="c", subcore_axis_name="s")

    @functools.partial(
        pl.kernel,
        mesh=mesh,
        out_type=jax.ShapeDtypeStruct((NW, NCHUNK, CHUNK, D), jnp.float32),
        scratch_types=[
            pltpu.VMEM((NCHUNK, CHUNK), jnp.int32),
            pltpu.VMEM((CHUNK, D), jnp.float32),
            pltpu.VMEM((CHUNK, D), jnp.float32),
            pltpu.SemaphoreType.DMA,
            pltpu.SemaphoreType.DMA,
        ],
    )
    def gather_k(table_hbm, idx_hbm, out_hbm, idx_v, buf0, buf1, sem0, sem1):
        wid = lax.axis_index("s") * 2 + lax.axis_index("c")
        pltpu.sync_copy(idx_hbm.at[wid], idx_v)

        def body(i, carry):
            j0 = 2 * i
            j1 = j0 + 1
            c0 = pltpu.async_copy(table_hbm.at[idx_v.at[j0]], buf0, sem0)
            c1 = pltpu.async_copy(table_hbm.at[idx_v.at[j1]], buf1, sem1)
            c0.wait()
            pltpu.sync_copy(buf0, out_hbm.at[wid, j0])
            c1.wait()
            pltpu.sync_copy(buf1, out_hbm.at[wid, j1])
            return carry

        lax.fori_loop(0, NCHUNK // 2, body, 0)

    return gather_k


def _sc_gather(table, idx_r):
    return _sc_gather_kernel()(table, idx_r)


@functools.cache
def _sc_scatter_kernel():
    mesh = plsc.VectorSubcoreMesh(core_axis_name="c", subcore_axis_name="s")

    @functools.partial(
        pl.kernel,
        mesh=mesh,
        out_type=jax.ShapeDtypeStruct((2, N_PAD, D), jnp.float32),
        scratch_types=[
            pltpu.VMEM((NCHUNK, CHUNK), jnp.int32),
            pltpu.VMEM((CHUNK, D), jnp.float32),
            pltpu.VMEM((CHUNK, D), jnp.float32),
            pltpu.VMEM_SHARED((N_PAD, D), jnp.float32),
            pltpu.SemaphoreType.DMA,
            pltpu.SemaphoreType.DMA,
        ],
    )
    def scatter_k(m_hbm, idx_hbm, z_hbm, out_hbm,
                  idx_v, buf0, buf1, acc, sem0, sem1):
        c = lax.axis_index("c")
        s = lax.axis_index("s")
        wid = s * 2 + c
        rows_per_sub = N_PAD // 16

        pltpu.sync_copy(z_hbm, buf0)

        def zbody(t, carry):
            pltpu.sync_copy(
                buf0, acc.at[pl.ds(s * rows_per_sub + t * CHUNK, CHUNK)])
            return carry

        lax.fori_loop(0, rows_per_sub // CHUNK, zbody, 0)
        pltpu.sync_copy(idx_hbm.at[wid], idx_v)
        plsc.subcore_barrier()

        def body(i, carry):
            j0 = 2 * i
            j1 = j0 + 1
            c0 = pltpu.async_copy(m_hbm.at[wid, j0], buf0, sem0)
            c1 = pltpu.async_copy(m_hbm.at[wid, j1], buf1, sem1)
            c0.wait()
            pltpu.sync_copy(buf0, acc.at[idx_v.at[j0]], add=True)
            c1.wait()
            pltpu.sync_copy(buf1, acc.at[idx_v.at[j1]], add=True)
            return carry

        lax.fori_loop(0, NCHUNK // 2, body, 0)
        plsc.subcore_barrier()

        pltpu.sync_copy(acc.at[pl.ds(s * rows_per_sub, rows_per_sub)],
                        out_hbm.at[c, pl.ds(s * rows_per_sub, rows_per_sub)])

    return scatter_k


def _sc_scatter(m_r, idx_r, zeros_blk):
    return _sc_scatter_kernel()(m_r, idx_r, zeros_blk)


def _full(shape):
    return pl.BlockSpec(shape, lambda i: (0,) * len(shape))


def _node_tc(x, mp, ln_g, ln_b, w1h_msg):
    def body(x_ref, w1, b1, w2, b2, w3, b3, g, b, wm, h_ref, p_ref):
        xv = jnp.nan_to_num(x_ref[...], nan=0.0, posinf=0.0, neginf=0.0)
        h = jnp.maximum(xv @ w1[...] + b1[...], 0.0)
        h = jnp.maximum(h @ w2[...] + b2[...], 0.0)
        h = h @ w3[...] + b3[...]
        mu = jnp.mean(h, axis=-1, keepdims=True)
        var = jnp.mean((h - mu) ** 2, axis=-1, keepdims=True)
        hn = (h - mu) * lax.rsqrt(var + 1e-5) * g[...] + b[...]
        h_ref[...] = hn
        p_ref[...] = hn @ wm[...]

    return pl.pallas_call(
        body,
        grid=(N // NB,),
        in_specs=[
            pl.BlockSpec((NB, D), lambda i: (i, 0)),
            _full((D, D)), _full((1, D)), _full((D, D)), _full((1, D)),
            _full((D, D)), _full((1, D)), _full((1, D)), _full((1, D)),
            _full((D, D)),
        ],
        out_specs=[
            pl.BlockSpec((NB, D), lambda i: (i, 0)),
            pl.BlockSpec((NB, D), lambda i: (i, 0)),
        ],
        out_shape=[
            jax.ShapeDtypeStruct((N, D), jnp.float32),
            jax.ShapeDtypeStruct((N, D), jnp.float32),
        ],
    )(x, mp['W1'], mp['b1'].reshape(1, D), mp['W2'], mp['b2'].reshape(1, D),
      mp['W3'], mp['b3'].reshape(1, D), ln_g.reshape(1, D), ln_b.reshape(1, D),
      w1h_msg)


def _ea_tc(ea_pad, w1e, b1):
    def body(ea_ref, w_ref, b_ref, out_ref):
        ea = jnp.nan_to_num(ea_ref[...], nan=0.0, posinf=0.0, neginf=0.0)
        out_ref[...] = ea @ w_ref[...] + b_ref[...]

    return pl.pallas_call(
        body,
        grid=(E_PAD // EB,),
        in_specs=[
            pl.BlockSpec((EB, EDGE_DIM), lambda i: (i, 0)),
            _full((EDGE_DIM, D)), _full((1, D)),
        ],
        out_specs=pl.BlockSpec((EB, D), lambda i: (i, 0)),
        out_shape=jax.ShapeDtypeStruct((E_PAD, D), jnp.float32),
    )(ea_pad, w1e, b1.reshape(1, D))


def _msg_tc(g_arr, ea_arr, w2, b2, w3, b3):
    def body(g_ref, ea_ref, w2r, b2r, w3r, b3r, m_ref):
        h = jnp.maximum(g_ref[...] + ea_ref[...], 0.0)
        h = jnp.maximum(h @ w2r[...] + b2r[...], 0.0)
        m_ref[...] = h @ w3r[...] + b3r[...]

    return pl.pallas_call(
        body,
        grid=(E_PAD // EB,),
        in_specs=[
            pl.BlockSpec((EB, D), lambda i: (i, 0)),
            pl.BlockSpec((EB, D), lambda i: (i, 0)),
            _full((D, D)), _full((1, D)), _full((D, D)), _full((1, D)),
        ],
        out_specs=pl.BlockSpec((EB, D), lambda i: (i, 0)),
        out_shape=jax.ShapeDtypeStruct((E_PAD, D), jnp.float32),
    )(g_arr, ea_arr, w2, b2.reshape(1, D), w3, b3.reshape(1, D))


def _update_tc(h, p0, p1, up, ln_g, ln_b, w1h_msg, compute_mean):
    nb = N // NB
    w1 = up['W1']

    def body(h_ref, p0_ref, p1_ref, w1h, w1a, b1, w2, b2, w3, b3, g, b, wm,
             *outs):
        agg = p0_ref[...] + p1_ref[...]
        hv = h_ref[...]
        u = jnp.maximum(hv @ w1h[...] + agg @ w1a[...] + b1[...], 0.0)
        u = jnp.maximum(u @ w2[...] + b2[...], 0.0)
        u = u @ w3[...] + b3[...]
        hh = hv + u
        mu = jnp.mean(hh, axis=-1, keepdims=True)
        var = jnp.mean((hh - mu) ** 2, axis=-1, keepdims=True)
        hn = (hh - mu) * lax.rsqrt(var + 1e-5) * g[...] + b[...]
        outs[0][...] = hn
        outs[1][...] = hn @ wm[...]
        if compute_mean:
            i = pl.program_id(0)
            gacc = outs[2]

            @pl.when(i == 0)
            def _():
                gacc[...] = jnp.zeros_like(gacc)

            gacc[...] += jnp.sum(hn, axis=0, keepdims=True)

            @pl.when(i == nb - 1)
            def _():
                gacc[...] = gacc[...] * (1.0 / N)

    out_specs = [
        pl.BlockSpec((NB, D), lambda i: (i, 0)),
        pl.BlockSpec((NB, D), lambda i: (i, 0)),
    ]
    out_shape = [
        jax.ShapeDtypeStruct((N, D), jnp.float32),
        jax.ShapeDtypeStruct((N, D), jnp.float32),
    ]
    if compute_mean:
        out_specs.append(_full((1, D)))
        out_shape.append(jax.ShapeDtypeStruct((1, D), jnp.float32))

    return pl.pallas_call(
        body,
        grid=(nb,),
        in_specs=[
            pl.BlockSpec((NB, D), lambda i: (i, 0)),
            pl.BlockSpec((NB, D), lambda i: (i, 0)),
            pl.BlockSpec((NB, D), lambda i: (i, 0)),
            _full((D, D)), _full((D, D)), _full((1, D)), _full((D, D)),
            _full((1, D)), _full((D, D)), _full((1, D)), _full((1, D)),
            _full((1, D)), _full((D, D)),
        ],
        out_specs=out_specs,
        out_shape=out_shape,
    )(h, p0, p1, w1[:D], w1[D:], up['b1'].reshape(1, D), up['W2'],
      up['b2'].reshape(1, D), up['W3'], up['b3'].reshape(1, D),
      ln_g.reshape(1, D), ln_b.reshape(1, D), w1h_msg)


def kernel(node_x, edge_index, edge_attr, params):
    node_x = node_x.astype(jnp.float32)
    edge_attr = edge_attr.astype(jnp.float32)
    src = edge_index[0].astype(jnp.int32)
    dst = edge_index[1].astype(jnp.int32)

    src_r = jnp.concatenate([src, jnp.zeros((PAD,), jnp.int32)]
                            ).reshape(NW, NCHUNK, CHUNK)
    dst_r = jnp.concatenate([dst, jnp.full((PAD,), N, jnp.int32)]
                            ).reshape(NW, NCHUNK, CHUNK)
    ea_pad = jnp.concatenate(
        [edge_attr, jnp.zeros((PAD, EDGE_DIM), jnp.float32)])

    mp = params['msg_mlp']
    w1h_msg = mp['W1'][:D]
    ln_g, ln_b = params['ln_g'], params['ln_b']

    EA = _ea_tc(ea_pad, mp['W1'][D:], mp['b1'])
    H, P = _node_tc(node_x, params['node_mlp'], ln_g, ln_b, w1h_msg)
    zeros_blk = jnp.zeros((CHUNK, D), jnp.float32)

    gsum = None
    for layer in range(N_LAYERS):
        G = _sc_gather(P, src_r).reshape(E_PAD, D)
        M = _msg_tc(G, EA, mp['W2'], mp['b2'], mp['W3'], mp['b3'])
        part = _sc_scatter(M.reshape(NW, NCHUNK, CHUNK, D), dst_r, zeros_blk)
        part = part[:, :N, :]
        last = layer == N_LAYERS - 1
        if last:
            H, P, gsum = _update_tc(H, part[0], part[1], params['up_mlp'],
                                    ln_g, ln_b, w1h_msg, True)
        else:
            H, P = _update_tc(H, part[0], part[1], params['up_mlp'],
                              ln_g, ln_b, w1h_msg, False)

    return (H, gsum.reshape(D))


# speedup vs baseline: 1.7869x; 1.4129x over previous
import functools

import jax
import jax.numpy as jnp
from jax import lax
from jax.experimental import pallas as pl
from jax.experimental.pallas import tpu as pltpu
from jax.experimental.pallas import tpu_sc as plsc

N = 10000
D = 128
EDGE_DIM = 16
E = 320000
N_LAYERS = 3

NW = 32
CHUNK = 128
NCHUNK = 80
EPW = NCHUNK * CHUNK
E_PAD = NW * EPW
PAD = E_PAD - E
N_PAD = 10240

NB = 400
EB = 512


@functools.cache
def _sc_gather_kernel():
    mesh = plsc.VectorSubcoreMesh(core_axis_name="c", subcore_axis_name="s")

    @functools.partial(
        pl.kernel,
        mesh=mesh,
        out_type=jax.ShapeDtypeStruct((NW, NCHUNK, CHUNK, D), jnp.float32),
        scratch_types=[
            pltpu.VMEM((NCHUNK, CHUNK), jnp.int32),
            pltpu.VMEM((CHUNK, D), jnp.float32),
            pltpu.VMEM((CHUNK, D), jnp.float32),
            pltpu.VMEM_SHARED((N, D), jnp.float32),
            pltpu.SemaphoreType.DMA,
            pltpu.SemaphoreType.DMA,
        ],
    )
    def gather_k(table_hbm, idx_hbm, out_hbm, idx_v, buf0, buf1, tbl,
                 sem0, sem1):
        c = lax.axis_index("c")
        s = lax.axis_index("s")
        wid = s * 2 + c

        pltpu.sync_copy(table_hbm.at[pl.ds(s * 624, 624)],
                        tbl.at[pl.ds(s * 624, 624)])

        @pl.when(s == 0)
        def _():
            pltpu.sync_copy(table_hbm.at[pl.ds(9984, 16)],
                            tbl.at[pl.ds(9984, 16)])

        pltpu.sync_copy(idx_hbm.at[wid], idx_v)
        plsc.subcore_barrier()

        def body(i, carry):
            j0 = 2 * i
            j1 = j0 + 1
            c0 = pltpu.async_copy(tbl.at[idx_v.at[j0]], buf0, sem0)
            c1 = pltpu.async_copy(tbl.at[idx_v.at[j1]], buf1, sem1)
            c0.wait()
            pltpu.sync_copy(buf0, out_hbm.at[wid, j0])
            c1.wait()
            pltpu.sync_copy(buf1, out_hbm.at[wid, j1])
            return carry

        lax.fori_loop(0, NCHUNK // 2, body, 0)

    return gather_k


def _sc_gather(table, idx_r):
    return _sc_gather_kernel()(table, idx_r)


@functools.cache
def _sc_scatter_kernel():
    mesh = plsc.VectorSubcoreMesh(core_axis_name="c", subcore_axis_name="s")

    @functools.partial(
        pl.kernel,
        mesh=mesh,
        out_type=jax.ShapeDtypeStruct((2, N_PAD, D), jnp.float32),
        scratch_types=[
            pltpu.VMEM((NCHUNK, CHUNK), jnp.int32),
            pltpu.VMEM((CHUNK, D), jnp.float32),
            pltpu.VMEM((CHUNK, D), jnp.float32),
            pltpu.VMEM_SHARED((N_PAD, D), jnp.float32),
            pltpu.SemaphoreType.DMA,
            pltpu.SemaphoreType.DMA,
        ],
    )
    def scatter_k(m_hbm, idx_hbm, z_hbm, out_hbm,
                  idx_v, buf0, buf1, acc, sem0, sem1):
        c = lax.axis_index("c")
        s = lax.axis_index("s")
        wid = s * 2 + c
        rows_per_sub = N_PAD // 16

        pltpu.sync_copy(z_hbm, buf0)

        def zbody(t, carry):
            pltpu.sync_copy(
                buf0, acc.at[pl.ds(s * rows_per_sub + t * CHUNK, CHUNK)])
            return carry

        lax.fori_loop(0, rows_per_sub // CHUNK, zbody, 0)
        pltpu.sync_copy(idx_hbm.at[wid], idx_v)
        plsc.subcore_barrier()

        def body(i, carry):
            j0 = 2 * i
            j1 = j0 + 1
            c0 = pltpu.async_copy(m_hbm.at[wid, j0], buf0, sem0)
            c1 = pltpu.async_copy(m_hbm.at[wid, j1], buf1, sem1)
            c0.wait()
            pltpu.sync_copy(buf0, acc.at[idx_v.at[j0]], add=True)
            c1.wait()
            pltpu.sync_copy(buf1, acc.at[idx_v.at[j1]], add=True)
            return carry

        lax.fori_loop(0, NCHUNK // 2, body, 0)
        plsc.subcore_barrier()

        pltpu.sync_copy(acc.at[pl.ds(s * rows_per_sub, rows_per_sub)],
                        out_hbm.at[c, pl.ds(s * rows_per_sub, rows_per_sub)])

    return scatter_k


def _sc_scatter(m_r, idx_r, zeros_blk):
    return _sc_scatter_kernel()(m_r, idx_r, zeros_blk)


def _full(shape):
    return pl.BlockSpec(shape, lambda i: (0,) * len(shape))


def _node_tc(x, mp, ln_g, ln_b, w1h_msg):
    def body(x_ref, w1, b1, w2, b2, w3, b3, g, b, wm, h_ref, p_ref):
        xv = jnp.nan_to_num(x_ref[...], nan=0.0, posinf=0.0, neginf=0.0)
        h = jnp.maximum(xv @ w1[...] + b1[...], 0.0)
        h = jnp.maximum(h @ w2[...] + b2[...], 0.0)
        h = h @ w3[...] + b3[...]
        mu = jnp.mean(h, axis=-1, keepdims=True)
        var = jnp.mean((h - mu) ** 2, axis=-1, keepdims=True)
        hn = (h - mu) * lax.rsqrt(var + 1e-5) * g[...] + b[...]
        h_ref[...] = hn
        p_ref[...] = hn @ wm[...]

    return pl.pallas_call(
        body,
        grid=(N // NB,),
        in_specs=[
            pl.BlockSpec((NB, D), lambda i: (i, 0)),
            _full((D, D)), _full((1, D)), _full((D, D)), _full((1, D)),
            _full((D, D)), _full((1, D)), _full((1, D)), _full((1, D)),
            _full((D, D)),
        ],
        out_specs=[
            pl.BlockSpec((NB, D), lambda i: (i, 0)),
            pl.BlockSpec((NB, D), lambda i: (i, 0)),
        ],
        out_shape=[
            jax.ShapeDtypeStruct((N, D), jnp.float32),
            jax.ShapeDtypeStruct((N, D), jnp.float32),
        ],
    )(x, mp['W1'], mp['b1'].reshape(1, D), mp['W2'], mp['b2'].reshape(1, D),
      mp['W3'], mp['b3'].reshape(1, D), ln_g.reshape(1, D), ln_b.reshape(1, D),
      w1h_msg)


def _ea_tc(ea_pad, w1e, b1):
    def body(ea_ref, w_ref, b_ref, out_ref):
        ea = jnp.nan_to_num(ea_ref[...], nan=0.0, posinf=0.0, neginf=0.0)
        out_ref[...] = ea @ w_ref[...] + b_ref[...]

    return pl.pallas_call(
        body,
        grid=(E_PAD // EB,),
        in_specs=[
            pl.BlockSpec((EB, EDGE_DIM), lambda i: (i, 0)),
            _full((EDGE_DIM, D)), _full((1, D)),
        ],
        out_specs=pl.BlockSpec((EB, D), lambda i: (i, 0)),
        out_shape=jax.ShapeDtypeStruct((E_PAD, D), jnp.float32),
    )(ea_pad, w1e, b1.reshape(1, D))


def _msg_tc(g_arr, ea_arr, w2, b2, w3, b3):
    def body(g_ref, ea_ref, w2r, b2r, w3r, b3r, m_ref):
        h = jnp.maximum(g_ref[...] + ea_ref[...], 0.0)
        h = jnp.maximum(h @ w2r[...] + b2r[...], 0.0)
        m_ref[...] = h @ w3r[...] + b3r[...]

    return pl.pallas_call(
        body,
        grid=(E_PAD // EB,),
        in_specs=[
            pl.BlockSpec((EB, D), lambda i: (i, 0)),
            pl.BlockSpec((EB, D), lambda i: (i, 0)),
            _full((D, D)), _full((1, D)), _full((D, D)), _full((1, D)),
        ],
        out_specs=pl.BlockSpec((EB, D), lambda i: (i, 0)),
        out_shape=jax.ShapeDtypeStruct((E_PAD, D), jnp.float32),
    )(g_arr, ea_arr, w2, b2.reshape(1, D), w3, b3.reshape(1, D))


def _update_tc(h, p0, p1, up, ln_g, ln_b, w1h_msg, compute_mean):
    nb = N // NB
    w1 = up['W1']

    def body(h_ref, p0_ref, p1_ref, w1h, w1a, b1, w2, b2, w3, b3, g, b, wm,
             *outs):
        agg = p0_ref[...] + p1_ref[...]
        hv = h_ref[...]
        u = jnp.maximum(hv @ w1h[...] + agg @ w1a[...] + b1[...], 0.0)
        u = jnp.maximum(u @ w2[...] + b2[...], 0.0)
        u = u @ w3[...] + b3[...]
        hh = hv + u
        mu = jnp.mean(hh, axis=-1, keepdims=True)
        var = jnp.mean((hh - mu) ** 2, axis=-1, keepdims=True)
        hn = (hh - mu) * lax.rsqrt(var + 1e-5) * g[...] + b[...]
        outs[0][...] = hn
        outs[1][...] = hn @ wm[...]
        if compute_mean:
            i = pl.program_id(0)
            gacc = outs[2]

            @pl.when(i == 0)
            def _():
                gacc[...] = jnp.zeros_like(gacc)

            gacc[...] += jnp.sum(hn, axis=0, keepdims=True)

            @pl.when(i == nb - 1)
            def _():
                gacc[...] = gacc[...] * (1.0 / N)

    out_specs = [
        pl.BlockSpec((NB, D), lambda i: (i, 0)),
        pl.BlockSpec((NB, D), lambda i: (i, 0)),
    ]
    out_shape = [
        jax.ShapeDtypeStruct((N, D), jnp.float32),
        jax.ShapeDtypeStruct((N, D), jnp.float32),
    ]
    if compute_mean:
        out_specs.append(_full((1, D)))
        out_shape.append(jax.ShapeDtypeStruct((1, D), jnp.float32))

    return pl.pallas_call(
        body,
        grid=(nb,),
        in_specs=[
            pl.BlockSpec((NB, D), lambda i: (i, 0)),
            pl.BlockSpec((NB, D), lambda i: (i, 0)),
            pl.BlockSpec((NB, D), lambda i: (i, 0)),
            _full((D, D)), _full((D, D)), _full((1, D)), _full((D, D)),
            _full((1, D)), _full((D, D)), _full((1, D)), _full((1, D)),
            _full((1, D)), _full((D, D)),
        ],
        out_specs=out_specs,
        out_shape=out_shape,
    )(h, p0, p1, w1[:D], w1[D:], up['b1'].reshape(1, D), up['W2'],
      up['b2'].reshape(1, D), up['W3'], up['b3'].reshape(1, D),
      ln_g.reshape(1, D), ln_b.reshape(1, D), w1h_msg)


def kernel(node_x, edge_index, edge_attr, params):
    node_x = node_x.astype(jnp.float32)
    edge_attr = edge_attr.astype(jnp.float32)
    src = edge_index[0].astype(jnp.int32)
    dst = edge_index[1].astype(jnp.int32)

    src_r = jnp.concatenate([src, jnp.zeros((PAD,), jnp.int32)]
                            ).reshape(NW, NCHUNK, CHUNK)
    dst_r = jnp.concatenate([dst, jnp.full((PAD,), N, jnp.int32)]
                            ).reshape(NW, NCHUNK, CHUNK)
    ea_pad = jnp.concatenate(
        [edge_attr, jnp.zeros((PAD, EDGE_DIM), jnp.float32)])

    mp = params['msg_mlp']
    w1h_msg = mp['W1'][:D]
    ln_g, ln_b = params['ln_g'], params['ln_b']

    EA = _ea_tc(ea_pad, mp['W1'][D:], mp['b1'])
    H, P = _node_tc(node_x, params['node_mlp'], ln_g, ln_b, w1h_msg)
    zeros_blk = jnp.zeros((CHUNK, D), jnp.float32)

    gsum = None
    for layer in range(N_LAYERS):
        G = _sc_gather(P, src_r).reshape(E_PAD, D)
        M = _msg_tc(G, EA, mp['W2'], mp['b2'], mp['W3'], mp['b3'])
        part = _sc_scatter(M.reshape(NW, NCHUNK, CHUNK, D), dst_r, zeros_blk)
        part = part[:, :N, :]
        last = layer == N_LAYERS - 1
        if last:
            H, P, gsum = _update_tc(H, part[0], part[1], params['up_mlp'],
                                    ln_g, ln_b, w1h_msg, True)
        else:
            H, P = _update_tc(H, part[0], part[1], params['up_mlp'],
                              ln_g, ln_b, w1h_msg, False)

    return (H, gsum.reshape(D))


# speedup vs baseline: 2.5028x; 1.4007x over previous
import functools

import jax
import jax.numpy as jnp
from jax import lax
from jax.experimental import pallas as pl
from jax.experimental.pallas import tpu as pltpu
from jax.experimental.pallas import tpu_sc as plsc

N = 10000
D = 128
EDGE_DIM = 16
E = 320000
N_LAYERS = 3

NW = 32
CHUNK = 128
NCHUNK = 80
EPW = NCHUNK * CHUNK
E_PAD = NW * EPW
PAD = E_PAD - E
N_PAD = 10240

NB = 400
EB = 800


@functools.cache
def _sc_gather_kernel():
    mesh = plsc.VectorSubcoreMesh(core_axis_name="c", subcore_axis_name="s")

    @functools.partial(
        pl.kernel,
        mesh=mesh,
        out_type=jax.ShapeDtypeStruct((NW, NCHUNK, CHUNK, D), jnp.float32),
        scratch_types=[
            pltpu.VMEM((NCHUNK, CHUNK), jnp.int32),
            pltpu.VMEM((CHUNK, D), jnp.float32),
            pltpu.VMEM((CHUNK, D), jnp.float32),
            pltpu.VMEM_SHARED((N, D), jnp.float32),
            pltpu.SemaphoreType.DMA,
            pltpu.SemaphoreType.DMA,
        ],
    )
    def gather_k(table_hbm, idx_hbm, out_hbm, idx_v, buf0, buf1, tbl,
                 sem0, sem1):
        c = lax.axis_index("c")
        s = lax.axis_index("s")
        wid = s * 2 + c

        pltpu.sync_copy(table_hbm.at[pl.ds(s * 624, 624)],
                        tbl.at[pl.ds(s * 624, 624)])

        @pl.when(s == 0)
        def _():
            pltpu.sync_copy(table_hbm.at[pl.ds(9984, 16)],
                            tbl.at[pl.ds(9984, 16)])

        pltpu.sync_copy(idx_hbm.at[wid], idx_v)
        plsc.subcore_barrier()

        def body(i, carry):
            j0 = 2 * i
            j1 = j0 + 1
            c0 = pltpu.async_copy(tbl.at[idx_v.at[j0]], buf0, sem0)
            c1 = pltpu.async_copy(tbl.at[idx_v.at[j1]], buf1, sem1)
            c0.wait()
            pltpu.sync_copy(buf0, out_hbm.at[wid, j0])
            c1.wait()
            pltpu.sync_copy(buf1, out_hbm.at[wid, j1])
            return carry

        lax.fori_loop(0, NCHUNK // 2, body, 0)

    return gather_k


def _sc_gather(table, idx_r):
    return _sc_gather_kernel()(table, idx_r)


@functools.cache
def _sc_scatter_kernel():
    mesh = plsc.VectorSubcoreMesh(core_axis_name="c", subcore_axis_name="s")

    @functools.partial(
        pl.kernel,
        mesh=mesh,
        out_type=jax.ShapeDtypeStruct((2, N_PAD, D), jnp.float32),
        scratch_types=[
            pltpu.VMEM((NCHUNK, CHUNK), jnp.int32),
            pltpu.VMEM((CHUNK, D), jnp.float32),
            pltpu.VMEM((CHUNK, D), jnp.float32),
            pltpu.VMEM_SHARED((N_PAD, D), jnp.float32),
            pltpu.SemaphoreType.DMA,
            pltpu.SemaphoreType.DMA,
        ],
    )
    def scatter_k(m_hbm, idx_hbm, z_hbm, out_hbm,
                  idx_v, buf0, buf1, acc, sem0, sem1):
        c = lax.axis_index("c")
        s = lax.axis_index("s")
        wid = s * 2 + c
        rows_per_sub = N_PAD // 16

        pltpu.sync_copy(z_hbm, buf0)

        def zbody(t, carry):
            pltpu.sync_copy(
                buf0, acc.at[pl.ds(s * rows_per_sub + t * CHUNK, CHUNK)])
            return carry

        lax.fori_loop(0, rows_per_sub // CHUNK, zbody, 0)
        pltpu.sync_copy(idx_hbm.at[wid], idx_v)
        plsc.subcore_barrier()

        def body(i, carry):
            j0 = 2 * i
            j1 = j0 + 1
            c0 = pltpu.async_copy(m_hbm.at[wid, j0], buf0, sem0)
            c1 = pltpu.async_copy(m_hbm.at[wid, j1], buf1, sem1)
            c0.wait()
            pltpu.sync_copy(buf0, acc.at[idx_v.at[j0]], add=True)
            c1.wait()
            pltpu.sync_copy(buf1, acc.at[idx_v.at[j1]], add=True)
            return carry

        lax.fori_loop(0, NCHUNK // 2, body, 0)
        plsc.subcore_barrier()

        pltpu.sync_copy(acc.at[pl.ds(s * rows_per_sub, rows_per_sub)],
                        out_hbm.at[c, pl.ds(s * rows_per_sub, rows_per_sub)])

    return scatter_k


def _sc_scatter(m_r, idx_r, zeros_blk):
    return _sc_scatter_kernel()(m_r, idx_r, zeros_blk)


def _full(shape):
    return pl.BlockSpec(shape, lambda i: (0,) * len(shape))


def _full2(shape):
    return pl.BlockSpec(shape, lambda i, c: (0,) * len(shape))


def _node_tc(x, mp, ln_g, ln_b, w1h_msg):
    def body(x_ref, w1, b1, w2, b2, w3, b3, g, b, wm, h_ref, p_ref):
        xv = jnp.nan_to_num(x_ref[...], nan=0.0, posinf=0.0, neginf=0.0)
        h = jnp.maximum(xv @ w1[...] + b1[...], 0.0)
        h = jnp.maximum(h @ w2[...] + b2[...], 0.0)
        h = h @ w3[...] + b3[...]
        mu = jnp.mean(h, axis=-1, keepdims=True)
        var = jnp.mean((h - mu) ** 2, axis=-1, keepdims=True)
        hn = (h - mu) * lax.rsqrt(var + 1e-5) * g[...] + b[...]
        h_ref[...] = hn
        p_ref[...] = hn @ wm[...]

    return pl.pallas_call(
        body,
        grid=(N // NB,),
        in_specs=[
            pl.BlockSpec((NB, D), lambda i: (i, 0)),
            _full((D, D)), _full((1, D)), _full((D, D)), _full((1, D)),
            _full((D, D)), _full((1, D)), _full((1, D)), _full((1, D)),
            _full((D, D)),
        ],
        out_specs=[
            pl.BlockSpec((NB, D), lambda i: (i, 0)),
            pl.BlockSpec((NB, D), lambda i: (i, 0)),
        ],
        out_shape=[
            jax.ShapeDtypeStruct((N, D), jnp.float32),
            jax.ShapeDtypeStruct((N, D), jnp.float32),
        ],
    )(x, mp['W1'], mp['b1'].reshape(1, D), mp['W2'], mp['b2'].reshape(1, D),
      mp['W3'], mp['b3'].reshape(1, D), ln_g.reshape(1, D), ln_b.reshape(1, D),
      w1h_msg)


def _msg_tc(g_arr, ea2, wbig, b1, w2, b2, w3, b3):
    E8 = E // 8
    ni = E8 // EB

    def body(g_ref, ea_ref, wb_ref, b1r, w2r, b2r, w3r, b3r, m_ref):
        c = pl.program_id(1)
        ea = jnp.nan_to_num(ea_ref[...], nan=0.0, posinf=0.0, neginf=0.0)
        wc = wb_ref[c]
        h = jnp.maximum(g_ref[...] + ea @ wc + b1r[...], 0.0)
        h = jnp.maximum(h @ w2r[...] + b2r[...], 0.0)
        m_ref[...] = h @ w3r[...] + b3r[...]

    return pl.pallas_call(
        body,
        grid=(ni, 8),
        in_specs=[
            pl.BlockSpec((EB, D), lambda i, c: (c * ni + i, 0)),
            pl.BlockSpec((EB, D), lambda i, c: (i, 0)),
            pl.BlockSpec((8, D, D), lambda i, c: (0, 0, 0)),
            _full2((1, D)), _full2((D, D)), _full2((1, D)),
            _full2((D, D)), _full2((1, D)),
        ],
        out_specs=pl.BlockSpec((EB, D), lambda i, c: (c * ni + i, 0)),
        out_shape=jax.ShapeDtypeStruct((E_PAD, D), jnp.float32),
    )(g_arr, ea2, wbig, b1.reshape(1, D), w2, b2.reshape(1, D), w3,
      b3.reshape(1, D))


def _update_tc(h, part, up, ln_g, ln_b, w1h_msg, compute_mean):
    nb = N // NB
    w1 = up['W1']

    def body(h_ref, p0_ref, p1_ref, w1h, w1a, b1, w2, b2, w3, b3, g, b, wm,
             *outs):
        agg = p0_ref[0] + p1_ref[0]
        hv = h_ref[...]
        u = jnp.maximum(hv @ w1h[...] + agg @ w1a[...] + b1[...], 0.0)
        u = jnp.maximum(u @ w2[...] + b2[...], 0.0)
        u = u @ w3[...] + b3[...]
        hh = hv + u
        mu = jnp.mean(hh, axis=-1, keepdims=True)
        var = jnp.mean((hh - mu) ** 2, axis=-1, keepdims=True)
        hn = (hh - mu) * lax.rsqrt(var + 1e-5) * g[...] + b[...]
        outs[0][...] = hn
        outs[1][...] = hn @ wm[...]
        if compute_mean:
            i = pl.program_id(0)
            gacc = outs[2]

            @pl.when(i == 0)
            def _():
                gacc[...] = jnp.zeros_like(gacc)

            gacc[...] += jnp.sum(hn, axis=0, keepdims=True)

            @pl.when(i == nb - 1)
            def _():
                gacc[...] = gacc[...] * (1.0 / N)

    out_specs = [
        pl.BlockSpec((NB, D), lambda i: (i, 0)),
        pl.BlockSpec((NB, D), lambda i: (i, 0)),
    ]
    out_shape = [
        jax.ShapeDtypeStruct((N, D), jnp.float32),
        jax.ShapeDtypeStruct((N, D), jnp.float32),
    ]
    if compute_mean:
        out_specs.append(_full((1, D)))
        out_shape.append(jax.ShapeDtypeStruct((1, D), jnp.float32))

    return pl.pallas_call(
        body,
        grid=(nb,),
        in_specs=[
            pl.BlockSpec((NB, D), lambda i: (i, 0)),
            pl.BlockSpec((1, NB, D), lambda i: (0, i, 0)),
            pl.BlockSpec((1, NB, D), lambda i: (1, i, 0)),
            _full((D, D)), _full((D, D)), _full((1, D)), _full((D, D)),
            _full((1, D)), _full((D, D)), _full((1, D)), _full((1, D)),
            _full((1, D)), _full((D, D)),
        ],
        out_specs=out_specs,
        out_shape=out_shape,
    )(h, part, part, w1[:D], w1[D:], up['b1'].reshape(1, D), up['W2'],
      up['b2'].reshape(1, D), up['W3'], up['b3'].reshape(1, D),
      ln_g.reshape(1, D), ln_b.reshape(1, D), w1h_msg)


def kernel(node_x, edge_index, edge_attr, params):
    node_x = node_x.astype(jnp.float32)
    edge_attr = edge_attr.astype(jnp.float32)
    src = edge_index[0].astype(jnp.int32)
    dst = edge_index[1].astype(jnp.int32)

    E8 = E // 8
    srcp = src.reshape(E8, 8).T.reshape(-1)
    dstp = dst.reshape(E8, 8).T.reshape(-1)
    ea2 = edge_attr.reshape(E8, 8 * EDGE_DIM)

    src_r = jnp.concatenate([srcp, jnp.zeros((PAD,), jnp.int32)]
                            ).reshape(NW, NCHUNK, CHUNK)
    dst_r = jnp.concatenate([dstp, jnp.full((PAD,), N, jnp.int32)]
                            ).reshape(NW, NCHUNK, CHUNK)

    mp = params['msg_mlp']
    w1h_msg = mp['W1'][:D]
    w1e = mp['W1'][D:]
    wbig = jnp.zeros((8, D, D), jnp.float32)
    for c in range(8):
        wbig = wbig.at[c, 16 * c:16 * (c + 1), :].set(w1e)
    ln_g, ln_b = params['ln_g'], params['ln_b']

    H, P = _node_tc(node_x, params['node_mlp'], ln_g, ln_b, w1h_msg)
    zeros_blk = jnp.zeros((CHUNK, D), jnp.float32)

    gsum = None
    for layer in range(N_LAYERS):
        G = _sc_gather(P, src_r).reshape(E_PAD, D)
        M = _msg_tc(G, ea2, wbig, mp['b1'], mp['W2'], mp['b2'],
                    mp['W3'], mp['b3'])
        part = _sc_scatter(M.reshape(NW, NCHUNK, CHUNK, D), dst_r, zeros_blk)
        last = layer == N_LAYERS - 1
        if last:
            H, P, gsum = _update_tc(H, part, params['up_mlp'],
                                    ln_g, ln_b, w1h_msg, True)
        else:
            H, P = _update_tc(H, part, params['up_mlp'],
                              ln_g, ln_b, w1h_msg, False)

    return (H, gsum.reshape(D))
